# baseline (speedup 1.0000x reference)
import functools

import jax
import jax.numpy as jnp
from jax.experimental import pallas as pl
from jax.experimental.pallas import tpu as pltpu

S, D = 2048, 1024
H, DK = 16, 64
E, DFF = 8, 2048
BS = 128
NB = S // BS + E - 1
PAD_T = NB * BS
_HI = jax.lax.Precision.HIGHEST


def _k1_ln_qkv(x_ref, g_ref, b_ref, wq_ref, bq_ref, wk_ref, bk_ref,
               wv_ref, bv_ref, q_ref, k_ref, v_ref):
    xb = x_ref[...]
    mu = jnp.mean(xb, axis=1, keepdims=True)
    var = jnp.mean((xb - mu) ** 2, axis=1, keepdims=True)
    z = (xb - mu) * jax.lax.rsqrt(var + 1e-5) * g_ref[...] + b_ref[...]
    q_ref[...] = jnp.dot(z, wq_ref[...], precision=_HI) + bq_ref[...]
    k_ref[...] = jnp.dot(z, wk_ref[...], precision=_HI) + bk_ref[...]
    v_ref[...] = jnp.dot(z, wv_ref[...], precision=_HI) + bv_ref[...]


def _k2_attn(q_ref, k_ref, v_ref, o_ref):
    q = q_ref[0]
    s = jax.lax.dot_general(q, k_ref[0], (((1,), (1,)), ((), ())),
                            precision=_HI) * (1.0 / 8.0)
    m = jnp.max(s, axis=1, keepdims=True)
    p = jnp.exp(s - m)
    l = jnp.sum(p, axis=1, keepdims=True)
    o_ref[0] = jnp.dot(p, v_ref[0], precision=_HI) / l


def _k3_post(x_ref, c_ref, wo_ref, bo_ref, g_ref, b_ref, ws_ref, bs_ref,
             x2_ref, z2_ref, lg_ref):
    x2 = x_ref[...] + jnp.dot(c_ref[...], wo_ref[...], precision=_HI) + bo_ref[...]
    x2_ref[...] = x2
    mu = jnp.mean(x2, axis=1, keepdims=True)
    var = jnp.mean((x2 - mu) ** 2, axis=1, keepdims=True)
    z2 = (x2 - mu) * jax.lax.rsqrt(var + 1e-5) * g_ref[...] + b_ref[...]
    z2_ref[...] = z2
    lg_ref[...] = jnp.dot(z2, ws_ref[...], precision=_HI) + bs_ref[...]


def _k5_ffn(e_ref, z_ref, w1_ref, b1_ref, w2_ref, b2_ref, y_ref):
    z = z_ref[...].astype(jnp.bfloat16)
    h = jnp.dot(z, w1_ref[0], preferred_element_type=jnp.float32) + b1_ref[0]
    h = jnp.maximum(h, 0.0).astype(jnp.bfloat16)
    y_ref[...] = jnp.dot(h, w2_ref[0], preferred_element_type=jnp.float32) + b2_ref[0]


def kernel(x, mask, ln1_g, ln1_b, ln2_g, ln2_b, Wq, bq, Wk, bk, Wv, bv,
           Wo, bo, Ws, bs, W1, b1, W2, b2):
    x2d = x.reshape(S, D)
    g1 = ln1_g.reshape(1, D); b1v = ln1_b.reshape(1, D)
    g2 = ln2_g.reshape(1, D); b2v = ln2_b.reshape(1, D)
    bq2 = bq.reshape(1, D); bk2 = bk.reshape(1, D)
    bv2 = bv.reshape(1, D); bo2 = bo.reshape(1, D)
    ws_pad = jnp.zeros((D, 128), jnp.float32).at[:, :E].set(Ws)
    bs_pad = jnp.zeros((1, 128), jnp.float32).at[0, :E].set(bs)

    BT = 256
    qkv = pl.pallas_call(
        _k1_ln_qkv,
        grid=(S // BT,),
        in_specs=[
            pl.BlockSpec((BT, D), lambda i: (i, 0)),
            pl.BlockSpec((1, D), lambda i: (0, 0)),
            pl.BlockSpec((1, D), lambda i: (0, 0)),
            pl.BlockSpec((D, D), lambda i: (0, 0)),
            pl.BlockSpec((1, D), lambda i: (0, 0)),
            pl.BlockSpec((D, D), lambda i: (0, 0)),
            pl.BlockSpec((1, D), lambda i: (0, 0)),
            pl.BlockSpec((D, D), lambda i: (0, 0)),
            pl.BlockSpec((1, D), lambda i: (0, 0)),
        ],
        out_specs=[pl.BlockSpec((BT, D), lambda i: (i, 0))] * 3,
        out_shape=[jax.ShapeDtypeStruct((S, D), jnp.float32)] * 3,
    )(x2d, g1, b1v, Wq, bq2, Wk, bk2, Wv, bv2)
    q, k, v = qkv

    qh = q.reshape(S, H, DK).transpose(1, 0, 2)
    kh = k.reshape(S, H, DK).transpose(1, 0, 2)
    vh = v.reshape(S, H, DK).transpose(1, 0, 2)
    ctx3 = pl.pallas_call(
        _k2_attn,
        grid=(H, S // BT),
        in_specs=[
            pl.BlockSpec((1, BT, DK), lambda h, i: (h, i, 0)),
            pl.BlockSpec((1, S, DK), lambda h, i: (h, 0, 0)),
            pl.BlockSpec((1, S, DK), lambda h, i: (h, 0, 0)),
        ],
        out_specs=pl.BlockSpec((1, BT, DK), lambda h, i: (h, i, 0)),
        out_shape=jax.ShapeDtypeStruct((H, S, DK), jnp.float32),
    )(qh, kh, vh)
    ctx = ctx3.transpose(1, 0, 2).reshape(S, D)

    x2, z2, lg_pad = pl.pallas_call(
        _k3_post,
        grid=(S // BT,),
        in_specs=[
            pl.BlockSpec((BT, D), lambda i: (i, 0)),
            pl.BlockSpec((BT, D), lambda i: (i, 0)),
            pl.BlockSpec((D, D), lambda i: (0, 0)),
            pl.BlockSpec((1, D), lambda i: (0, 0)),
            pl.BlockSpec((1, D), lambda i: (0, 0)),
            pl.BlockSpec((1, D), lambda i: (0, 0)),
            pl.BlockSpec((D, 128), lambda i: (0, 0)),
            pl.BlockSpec((1, 128), lambda i: (0, 0)),
        ],
        out_specs=[
            pl.BlockSpec((BT, D), lambda i: (i, 0)),
            pl.BlockSpec((BT, D), lambda i: (i, 0)),
            pl.BlockSpec((BT, 128), lambda i: (i, 0)),
        ],
        out_shape=[
            jax.ShapeDtypeStruct((S, D), jnp.float32),
            jax.ShapeDtypeStruct((S, D), jnp.float32),
            jax.ShapeDtypeStruct((S, 128), jnp.float32),
        ],
    )(x2d, ctx, Wo, bo2, g2, b2v, ws_pad, bs_pad)

    logits = lg_pad[:, :E]

    m = jnp.max(logits, axis=1)
    ex = jnp.exp(logits - m[:, None])
    sden = jnp.sum(ex, axis=1)
    rpm = 1.0 / sden
    probs = ex * rpm[:, None]
    route = jnp.argmax(logits, axis=1).astype(jnp.int32)
    counts = jnp.sum(route[:, None] == jnp.arange(E)[None, :], axis=0).astype(jnp.int32)
    nb = (counts + BS - 1) // BS
    cumnb = jnp.cumsum(nb)
    pstart = (cumnb - nb) * BS
    sort_idx = jnp.argsort(route, stable=True).astype(jnp.int32)
    r_sorted = route[sort_idx]
    rank = jnp.arange(S, dtype=jnp.int32) - (jnp.cumsum(counts) - counts)[r_sorted]
    pos_sorted = pstart[r_sorted] + rank
    pos = jnp.zeros((S,), jnp.int32).at[sort_idx].set(pos_sorted)
    z_sorted = jnp.zeros((PAD_T, D), jnp.float32).at[pos].set(z2)
    block_expert = jnp.minimum(
        jnp.sum(jnp.arange(NB, dtype=jnp.int32)[:, None] >= cumnb[None, :], axis=1),
        E - 1).astype(jnp.int32)

    w1b = W1.astype(jnp.bfloat16)
    w2b = W2.astype(jnp.bfloat16)
    y_sorted = pl.pallas_call(
        _k5_ffn,
        grid_spec=pltpu.PrefetchScalarGridSpec(
            num_scalar_prefetch=1,
            grid=(NB,),
            in_specs=[
                pl.BlockSpec((BS, D), lambda i, e: (i, 0)),
                pl.BlockSpec((1, D, DFF), lambda i, e: (e[i], 0, 0)),
                pl.BlockSpec((1, 1, DFF), lambda i, e: (e[i], 0, 0)),
                pl.BlockSpec((1, DFF, D), lambda i, e: (e[i], 0, 0)),
                pl.BlockSpec((1, 1, D), lambda i, e: (e[i], 0, 0)),
            ],
            out_specs=pl.BlockSpec((BS, D), lambda i, e: (i, 0)),
        ),
        out_shape=jax.ShapeDtypeStruct((PAD_T, D), jnp.float32),
    )(block_expert, z_sorted, w1b, b1.reshape(E, 1, DFF), w2b, b2.reshape(E, 1, D))

    out2d = x2 + rpm[:, None] * y_sorted[pos]
    out = out2d.reshape(S, 1, D)
    return (out, counts.astype(jnp.float32), jnp.sum(probs, axis=0),
            jnp.array(0, jnp.int32), rpm)


# speedup vs baseline: 2.4823x; 2.4823x over previous
import functools

import jax
import jax.numpy as jnp
from jax.experimental import pallas as pl
from jax.experimental.pallas import tpu as pltpu

S, D = 2048, 1024
H, DK = 16, 64
E, DFF = 8, 2048
BS = 128
NB = S // BS + E - 1
PAD_T = NB * BS


def _k1_ln_qkv(x_ref, g_ref, b_ref, wq_ref, bq_ref, wk_ref, bk_ref,
               wv_ref, bv_ref, q_ref, k_ref, v_ref):
    xb = x_ref[...]
    mu = jnp.mean(xb, axis=1, keepdims=True)
    var = jnp.mean((xb - mu) ** 2, axis=1, keepdims=True)
    z = (xb - mu) * jax.lax.rsqrt(var + 1e-5) * g_ref[...] + b_ref[...]
    q_ref[...] = jnp.dot(z, wq_ref[...]) + bq_ref[...]
    k_ref[...] = jnp.dot(z, wk_ref[...]) + bk_ref[...]
    v_ref[...] = jnp.dot(z, wv_ref[...]) + bv_ref[...]


def _k2_attn(q_ref, k_ref, v_ref, o_ref):
    q = q_ref[0]
    s = jax.lax.dot_general(q, k_ref[0],
                            (((1,), (1,)), ((), ()))) * (1.0 / 8.0)
    m = jnp.max(s, axis=1, keepdims=True)
    p = jnp.exp(s - m)
    l = jnp.sum(p, axis=1, keepdims=True)
    o_ref[0] = jnp.dot(p, v_ref[0]) / l


def _k3_post(x_ref, c_ref, wo_ref, bo_ref, g_ref, b_ref, ws_ref, bs_ref,
             x2_ref, z2_ref, lg_ref):
    x2 = x_ref[...] + jnp.dot(c_ref[...], wo_ref[...]) + bo_ref[...]
    x2_ref[...] = x2
    mu = jnp.mean(x2, axis=1, keepdims=True)
    var = jnp.mean((x2 - mu) ** 2, axis=1, keepdims=True)
    z2 = (x2 - mu) * jax.lax.rsqrt(var + 1e-5) * g_ref[...] + b_ref[...]
    z2_ref[...] = z2
    lg_ref[...] = jnp.dot(z2, ws_ref[...]) + bs_ref[...]


def _k5_ffn(e_ref, z_ref, w1_ref, b1_ref, w2_ref, b2_ref, y_ref):
    z = z_ref[...].astype(jnp.bfloat16)
    h = jnp.dot(z, w1_ref[0], preferred_element_type=jnp.float32) + b1_ref[0]
    h = jnp.maximum(h, 0.0).astype(jnp.bfloat16)
    y_ref[...] = jnp.dot(h, w2_ref[0], preferred_element_type=jnp.float32) + b2_ref[0]


def kernel(x, mask, ln1_g, ln1_b, ln2_g, ln2_b, Wq, bq, Wk, bk, Wv, bv,
           Wo, bo, Ws, bs, W1, b1, W2, b2):
    x2d = x.reshape(S, D)
    g1 = ln1_g.reshape(1, D); b1v = ln1_b.reshape(1, D)
    g2 = ln2_g.reshape(1, D); b2v = ln2_b.reshape(1, D)
    bq2 = bq.reshape(1, D); bk2 = bk.reshape(1, D)
    bv2 = bv.reshape(1, D); bo2 = bo.reshape(1, D)
    ws_pad = jnp.zeros((D, 128), jnp.float32).at[:, :E].set(Ws)
    bs_pad = jnp.zeros((1, 128), jnp.float32).at[0, :E].set(bs)

    BT = 256
    qkv = pl.pallas_call(
        _k1_ln_qkv,
        grid=(S // BT,),
        in_specs=[
            pl.BlockSpec((BT, D), lambda i: (i, 0)),
            pl.BlockSpec((1, D), lambda i: (0, 0)),
            pl.BlockSpec((1, D), lambda i: (0, 0)),
            pl.BlockSpec((D, D), lambda i: (0, 0)),
            pl.BlockSpec((1, D), lambda i: (0, 0)),
            pl.BlockSpec((D, D), lambda i: (0, 0)),
            pl.BlockSpec((1, D), lambda i: (0, 0)),
            pl.BlockSpec((D, D), lambda i: (0, 0)),
            pl.BlockSpec((1, D), lambda i: (0, 0)),
        ],
        out_specs=[pl.BlockSpec((BT, D), lambda i: (i, 0))] * 3,
        out_shape=[jax.ShapeDtypeStruct((S, D), jnp.float32)] * 3,
    )(x2d, g1, b1v, Wq, bq2, Wk, bk2, Wv, bv2)
    q, k, v = qkv

    qh = q.reshape(S, H, DK).transpose(1, 0, 2)
    kh = k.reshape(S, H, DK).transpose(1, 0, 2)
    vh = v.reshape(S, H, DK).transpose(1, 0, 2)
    ctx3 = pl.pallas_call(
        _k2_attn,
        grid=(H, S // BT),
        in_specs=[
            pl.BlockSpec((1, BT, DK), lambda h, i: (h, i, 0)),
            pl.BlockSpec((1, S, DK), lambda h, i: (h, 0, 0)),
            pl.BlockSpec((1, S, DK), lambda h, i: (h, 0, 0)),
        ],
        out_specs=pl.BlockSpec((1, BT, DK), lambda h, i: (h, i, 0)),
        out_shape=jax.ShapeDtypeStruct((H, S, DK), jnp.float32),
    )(qh, kh, vh)
    ctx = ctx3.transpose(1, 0, 2).reshape(S, D)

    x2, z2, lg_pad = pl.pallas_call(
        _k3_post,
        grid=(S // BT,),
        in_specs=[
            pl.BlockSpec((BT, D), lambda i: (i, 0)),
            pl.BlockSpec((BT, D), lambda i: (i, 0)),
            pl.BlockSpec((D, D), lambda i: (0, 0)),
            pl.BlockSpec((1, D), lambda i: (0, 0)),
            pl.BlockSpec((1, D), lambda i: (0, 0)),
            pl.BlockSpec((1, D), lambda i: (0, 0)),
            pl.BlockSpec((D, 128), lambda i: (0, 0)),
            pl.BlockSpec((1, 128), lambda i: (0, 0)),
        ],
        out_specs=[
            pl.BlockSpec((BT, D), lambda i: (i, 0)),
            pl.BlockSpec((BT, D), lambda i: (i, 0)),
            pl.BlockSpec((BT, 128), lambda i: (i, 0)),
        ],
        out_shape=[
            jax.ShapeDtypeStruct((S, D), jnp.float32),
            jax.ShapeDtypeStruct((S, D), jnp.float32),
            jax.ShapeDtypeStruct((S, 128), jnp.float32),
        ],
    )(x2d, ctx, Wo, bo2, g2, b2v, ws_pad, bs_pad)

    logits = lg_pad[:, :E]

    m = jnp.max(logits, axis=1)
    ex = jnp.exp(logits - m[:, None])
    sden = jnp.sum(ex, axis=1)
    rpm = 1.0 / sden
    probs = ex * rpm[:, None]
    route = jnp.argmax(logits, axis=1).astype(jnp.int32)
    counts = jnp.sum(route[:, None] == jnp.arange(E)[None, :], axis=0).astype(jnp.int32)
    nb = (counts + BS - 1) // BS
    cumnb = jnp.cumsum(nb)
    pstart = (cumnb - nb) * BS
    sort_idx = jnp.argsort(route, stable=True).astype(jnp.int32)
    r_sorted = route[sort_idx]
    rank = jnp.arange(S, dtype=jnp.int32) - (jnp.cumsum(counts) - counts)[r_sorted]
    pos_sorted = pstart[r_sorted] + rank
    pos = jnp.zeros((S,), jnp.int32).at[sort_idx].set(pos_sorted)
    z_sorted = jnp.zeros((PAD_T, D), jnp.float32).at[pos].set(z2)
    block_expert = jnp.minimum(
        jnp.sum(jnp.arange(NB, dtype=jnp.int32)[:, None] >= cumnb[None, :], axis=1),
        E - 1).astype(jnp.int32)

    w1b = W1.astype(jnp.bfloat16)
    w2b = W2.astype(jnp.bfloat16)
    y_sorted = pl.pallas_call(
        _k5_ffn,
        grid_spec=pltpu.PrefetchScalarGridSpec(
            num_scalar_prefetch=1,
            grid=(NB,),
            in_specs=[
                pl.BlockSpec((BS, D), lambda i, e: (i, 0)),
                pl.BlockSpec((1, D, DFF), lambda i, e: (e[i], 0, 0)),
                pl.BlockSpec((1, 1, DFF), lambda i, e: (e[i], 0, 0)),
                pl.BlockSpec((1, DFF, D), lambda i, e: (e[i], 0, 0)),
                pl.BlockSpec((1, 1, D), lambda i, e: (e[i], 0, 0)),
            ],
            out_specs=pl.BlockSpec((BS, D), lambda i, e: (i, 0)),
        ),
        out_shape=jax.ShapeDtypeStruct((PAD_T, D), jnp.float32),
    )(block_expert, z_sorted, w1b, b1.reshape(E, 1, DFF), w2b, b2.reshape(E, 1, D))

    out2d = x2 + rpm[:, None] * y_sorted[pos]
    out = out2d.reshape(S, 1, D)
    return (out, counts.astype(jnp.float32), jnp.sum(probs, axis=0),
            jnp.array(0, jnp.int32), rpm)


# speedup vs baseline: 2.6103x; 1.0515x over previous
import functools

import jax
import jax.numpy as jnp
from jax import lax
from jax.experimental import pallas as pl
from jax.experimental.pallas import tpu as pltpu
from jax.experimental.pallas import tpu_sc as plsc

S, D = 2048, 1024
H, DK = 16, 64
E, DFF = 8, 2048
BS = 128
NB = S // BS + E - 1
PAD_T = NB * BS
NW = 32
TPW = S // NW
_mesh = plsc.VectorSubcoreMesh(core_axis_name="c", subcore_axis_name="s")


def _k1_ln_qkv(x_ref, g_ref, b_ref, wq_ref, bq_ref, wk_ref, bk_ref,
               wv_ref, bv_ref, q_ref, k_ref, v_ref):
    xb = x_ref[...]
    mu = jnp.mean(xb, axis=1, keepdims=True)
    var = jnp.mean((xb - mu) ** 2, axis=1, keepdims=True)
    z = (xb - mu) * jax.lax.rsqrt(var + 1e-5) * g_ref[...] + b_ref[...]
    q_ref[...] = jnp.dot(z, wq_ref[...]) + bq_ref[...]
    k_ref[...] = jnp.dot(z, wk_ref[...]) + bk_ref[...]
    v_ref[...] = jnp.dot(z, wv_ref[...]) + bv_ref[...]


def _k2_attn(q_ref, k_ref, v_ref, o_ref):
    q = q_ref[0]
    s = jax.lax.dot_general(q, k_ref[0],
                            (((1,), (1,)), ((), ()))) * (1.0 / 8.0)
    m = jnp.max(s, axis=1, keepdims=True)
    p = jnp.exp(s - m)
    l = jnp.sum(p, axis=1, keepdims=True)
    o_ref[0] = jnp.dot(p, v_ref[0]) / l


def _k3_post(x_ref, c_ref, wo_ref, bo_ref, g_ref, b_ref, ws_ref, bs_ref,
             x2_ref, z2_ref, lg_ref):
    x2 = x_ref[...] + jnp.dot(c_ref[...], wo_ref[...]) + bo_ref[...]
    x2_ref[...] = x2
    mu = jnp.mean(x2, axis=1, keepdims=True)
    var = jnp.mean((x2 - mu) ** 2, axis=1, keepdims=True)
    z2 = (x2 - mu) * jax.lax.rsqrt(var + 1e-5) * g_ref[...] + b_ref[...]
    z2_ref[...] = z2
    lg_ref[...] = jnp.dot(z2, ws_ref[...]) + bs_ref[...]


def _dyng(x, idx):
    return lax.gather(
        x, idx[:, None],
        lax.GatherDimensionNumbers(offset_dims=(), collapsed_slice_dims=(0,),
                                   start_index_map=(0,)),
        (1,), mode=lax.GatherScatterMode.PROMISE_IN_BOUNDS)


def _prefix16(x, lane):
    zero = x - x
    for k in (1, 2, 4, 8):
        g = _dyng(x, jnp.maximum(lane - k, 0))
        x = x + jnp.where(lane >= k, g, zero)
    return x


@functools.partial(
    pl.kernel, mesh=_mesh,
    compiler_params=pltpu.CompilerParams(needs_layout_passes=False),
    out_type=(
        jax.ShapeDtypeStruct((S,), jnp.float32),
        jax.ShapeDtypeStruct((S,), jnp.int32),
        jax.ShapeDtypeStruct((NW * 16,), jnp.int32),
        jax.ShapeDtypeStruct((NW * 16,), jnp.float32),
    ),
    scratch_types=[
        pltpu.VMEM((E * TPW,), jnp.float32),
        pltpu.VMEM((TPW,), jnp.float32),
        pltpu.VMEM((TPW,), jnp.int32),
        pltpu.VMEM((16,), jnp.int32),
        pltpu.VMEM((16,), jnp.float32),
    ],
)
def _sc_router(lgT_hbm, rpm_hbm, route_hbm, cnt_hbm, ps_hbm,
               lg_v, rpm_v, rt_v, cnt_v, ps_v):
    wid = lax.axis_index("s") * 2 + lax.axis_index("c")
    base = wid * TPW
    pltpu.sync_copy(lgT_hbm.at[pl.ds(wid * E * TPW, E * TPW)], lg_v)
    lane = lax.iota(jnp.int32, 16)
    last = jnp.full((16,), 15, jnp.int32)
    cnt_acc = [jnp.zeros((16,), jnp.int32) for _ in range(E)]
    ps_acc = [jnp.zeros((16,), jnp.float32) for _ in range(E)]
    for c in range(TPW // 16):
        l = [lg_v[pl.ds(e * TPW + c * 16, 16)] for e in range(E)]
        m = l[0]
        for e in range(1, E):
            m = jnp.maximum(m, l[e])
        ex = [jnp.exp(l[e] - m) for e in range(E)]
        sden = ex[0]
        for e in range(1, E):
            sden = sden + ex[e]
        rpm_c = 1.0 / sden
        best = jnp.zeros((16,), jnp.int32)
        bl = l[0]
        for e in range(1, E):
            flip = l[e] > bl
            bl = jnp.maximum(bl, l[e])
            best = jnp.where(flip, e, best)
        rpm_v[pl.ds(c * 16, 16)] = rpm_c
        rt_v[pl.ds(c * 16, 16)] = best
        for e in range(E):
            ps_acc[e] = ps_acc[e] + ex[e] * rpm_c
            cnt_acc[e] = cnt_acc[e] + jnp.where(best == e, 1, 0)
    cntv = jnp.zeros((16,), jnp.int32)
    psv = jnp.zeros((16,), jnp.float32)
    for e in range(E):
        tot_c = _dyng(_prefix16(cnt_acc[e], lane), last)
        cntv = jnp.where(lane == e, tot_c, cntv)
        tot_p = _dyng(_prefix16(ps_acc[e], lane), last)
        psv = jnp.where(lane == e, tot_p, psv)
    cnt_v[...] = cntv
    ps_v[...] = psv
    pltpu.sync_copy(rpm_v, rpm_hbm.at[pl.ds(base, TPW)])
    pltpu.sync_copy(rt_v, route_hbm.at[pl.ds(base, TPW)])
    pltpu.sync_copy(cnt_v, cnt_hbm.at[pl.ds(wid * 16, 16)])
    pltpu.sync_copy(ps_v, ps_hbm.at[pl.ds(wid * 16, 16)])


@functools.partial(
    pl.kernel, mesh=_mesh,
    compiler_params=pltpu.CompilerParams(needs_layout_passes=False),
    out_type=(
        jax.ShapeDtypeStruct((S,), jnp.int32),
        jax.ShapeDtypeStruct((PAD_T, D), jnp.float32),
        jax.ShapeDtypeStruct((32,), jnp.int32),
        jax.ShapeDtypeStruct((16,), jnp.float32),
        jax.ShapeDtypeStruct((16,), jnp.float32),
    ),
    scratch_types=[
        pltpu.VMEM((NW * 16,), jnp.int32),
        pltpu.VMEM((NW * 16,), jnp.float32),
        pltpu.VMEM((TPW,), jnp.int32),
        pltpu.VMEM((TPW,), jnp.int32),
        pltpu.VMEM((32,), jnp.int32),
        pltpu.VMEM((16,), jnp.float32),
        pltpu.VMEM((TPW, D), jnp.float32),
        pltpu.SemaphoreType.DMA,
    ],
)
def _sc_dispatch(route_hbm, cnt_hbm, ps_hbm, z2_hbm,
                 pos_hbm, zs_hbm, be_hbm, cf_hbm, pso_hbm,
                 cp_v, pp_v, rt_v, pos_v, be_v, f_v, z_v, sem):
    wid = lax.axis_index("s") * 2 + lax.axis_index("c")
    base = wid * TPW
    pltpu.sync_copy(cnt_hbm, cp_v)
    pltpu.sync_copy(route_hbm.at[pl.ds(base, TPW)], rt_v)
    lane = lax.iota(jnp.int32, 16)
    last = jnp.full((16,), 15, jnp.int32)
    total = jnp.zeros((16,), jnp.int32)
    before = jnp.zeros((16,), jnp.int32)
    for t in range(NW):
        row = cp_v[pl.ds(t * 16, 16)]
        total = total + row
        before = before + row * jnp.where(wid > t, 1, 0)
    nb = lax.shift_right_logical(total + (BS - 1), 7)
    cumnb = _prefix16(nb, lane)
    pstart = (cumnb - nb) * BS
    runv = pstart + before
    for c in range(TPW // 16):
        r = rt_v[pl.ds(c * 16, 16)]
        posc = jnp.zeros((16,), jnp.int32)
        addv = jnp.zeros((16,), jnp.int32)
        for e in range(E):
            mk = r == e
            inc = _prefix16(mk.astype(jnp.int32), lane)
            tot = _dyng(inc, last)
            base_e = _dyng(runv, jnp.full((16,), e, jnp.int32))
            posc = jnp.where(mk, base_e + inc - 1, posc)
            addv = addv + jnp.where(lane == e, tot, 0)
        runv = runv + addv
        pos_v[pl.ds(c * 16, 16)] = posc
    pltpu.sync_copy(pos_v, pos_hbm.at[pl.ds(base, TPW)])
    pltpu.sync_copy(z2_hbm.at[pl.ds(base, TPW)], z_v)
    pltpu.async_copy(z_v, zs_hbm.at[pos_v], sem).wait()

    @pl.when(wid == 0)
    def _():
        acc0 = jnp.zeros((16,), jnp.int32)
        acc1 = jnp.zeros((16,), jnp.int32)
        for e in range(E):
            ce = _dyng(cumnb, jnp.full((16,), e, jnp.int32))
            acc0 = acc0 + jnp.where(ce <= lane, 1, 0)
            acc1 = acc1 + jnp.where(ce <= lane + 16, 1, 0)
        be_v[pl.ds(0, 16)] = jnp.minimum(acc0, E - 1)
        be_v[pl.ds(16, 16)] = jnp.minimum(acc1, E - 1)
        pltpu.sync_copy(be_v, be_hbm)
        f_v[...] = total.astype(jnp.float32)
        pltpu.sync_copy(f_v, cf_hbm)
        pltpu.sync_copy(ps_hbm, pp_v)
        pstot = jnp.zeros((16,), jnp.float32)
        for t in range(NW):
            pstot = pstot + pp_v[pl.ds(t * 16, 16)]
        f_v[...] = pstot
        pltpu.sync_copy(f_v, pso_hbm)


@functools.partial(
    pl.kernel, mesh=_mesh,
    compiler_params=pltpu.CompilerParams(needs_layout_passes=False),
    out_type=jax.ShapeDtypeStruct((S, D), jnp.float32),
    scratch_types=[
        pltpu.VMEM((32, D), jnp.float32),
        pltpu.VMEM((32, D), jnp.float32),
        pltpu.VMEM((32,), jnp.int32),
        pltpu.VMEM((32,), jnp.float32),
        pltpu.SemaphoreType.DMA,
    ],
)
def _sc_combine(x2_hbm, ys_hbm, pos_hbm, rpm_hbm, out_hbm,
                xv, yv, pv, rv, sem):
    wid = lax.axis_index("s") * 2 + lax.axis_index("c")
    for c in range(TPW // 32):
        b2 = wid * TPW + c * 32
        pltpu.sync_copy(x2_hbm.at[pl.ds(b2, 32)], xv)
        pltpu.sync_copy(pos_hbm.at[pl.ds(b2, 32)], pv)
        pltpu.sync_copy(rpm_hbm.at[pl.ds(b2, 32)], rv)
        pltpu.async_copy(ys_hbm.at[pv], yv, sem).wait()

        def body(t, carry):
            sc = plsc.load_gather(rv, [jnp.zeros((16,), jnp.int32) + t])
            for d in range(D // 16):
                sl = pl.ds(d * 16, 16)
                xv[t, sl] = xv[t, sl] + sc * yv[t, sl]
            return carry

        lax.fori_loop(0, 32, body, 0)
        pltpu.sync_copy(xv, out_hbm.at[pl.ds(b2, 32)])


def _k5_ffn(e_ref, z_ref, w1_ref, b1_ref, w2_ref, b2_ref, y_ref):
    z = z_ref[...].astype(jnp.bfloat16)
    h = jnp.dot(z, w1_ref[0], preferred_element_type=jnp.float32) + b1_ref[0]
    h = jnp.maximum(h, 0.0).astype(jnp.bfloat16)
    y_ref[...] = jnp.dot(h, w2_ref[0], preferred_element_type=jnp.float32) + b2_ref[0]


def kernel(x, mask, ln1_g, ln1_b, ln2_g, ln2_b, Wq, bq, Wk, bk, Wv, bv,
           Wo, bo, Ws, bs, W1, b1, W2, b2):
    x2d = x.reshape(S, D)
    g1 = ln1_g.reshape(1, D); b1v = ln1_b.reshape(1, D)
    g2 = ln2_g.reshape(1, D); b2v = ln2_b.reshape(1, D)
    bq2 = bq.reshape(1, D); bk2 = bk.reshape(1, D)
    bv2 = bv.reshape(1, D); bo2 = bo.reshape(1, D)
    ws_pad = jnp.zeros((D, 128), jnp.float32).at[:, :E].set(Ws)
    bs_pad = jnp.zeros((1, 128), jnp.float32).at[0, :E].set(bs)

    BT = 256
    qkv = pl.pallas_call(
        _k1_ln_qkv,
        grid=(S // BT,),
        in_specs=[
            pl.BlockSpec((BT, D), lambda i: (i, 0)),
            pl.BlockSpec((1, D), lambda i: (0, 0)),
            pl.BlockSpec((1, D), lambda i: (0, 0)),
            pl.BlockSpec((D, D), lambda i: (0, 0)),
            pl.BlockSpec((1, D), lambda i: (0, 0)),
            pl.BlockSpec((D, D), lambda i: (0, 0)),
            pl.BlockSpec((1, D), lambda i: (0, 0)),
            pl.BlockSpec((D, D), lambda i: (0, 0)),
            pl.BlockSpec((1, D), lambda i: (0, 0)),
        ],
        out_specs=[pl.BlockSpec((BT, D), lambda i: (i, 0))] * 3,
        out_shape=[jax.ShapeDtypeStruct((S, D), jnp.float32)] * 3,
    )(x2d, g1, b1v, Wq, bq2, Wk, bk2, Wv, bv2)
    q, k, v = qkv

    qh = q.reshape(S, H, DK).transpose(1, 0, 2)
    kh = k.reshape(S, H, DK).transpose(1, 0, 2)
    vh = v.reshape(S, H, DK).transpose(1, 0, 2)
    ctx3 = pl.pallas_call(
        _k2_attn,
        grid=(H, S // BT),
        in_specs=[
            pl.BlockSpec((1, BT, DK), lambda h, i: (h, i, 0)),
            pl.BlockSpec((1, S, DK), lambda h, i: (h, 0, 0)),
            pl.BlockSpec((1, S, DK), lambda h, i: (h, 0, 0)),
        ],
        out_specs=pl.BlockSpec((1, BT, DK), lambda h, i: (h, i, 0)),
        out_shape=jax.ShapeDtypeStruct((H, S, DK), jnp.float32),
    )(qh, kh, vh)
    ctx = ctx3.transpose(1, 0, 2).reshape(S, D)

    x2, z2, lg_pad = pl.pallas_call(
        _k3_post,
        grid=(S // BT,),
        in_specs=[
            pl.BlockSpec((BT, D), lambda i: (i, 0)),
            pl.BlockSpec((BT, D), lambda i: (i, 0)),
            pl.BlockSpec((D, D), lambda i: (0, 0)),
            pl.BlockSpec((1, D), lambda i: (0, 0)),
            pl.BlockSpec((1, D), lambda i: (0, 0)),
            pl.BlockSpec((1, D), lambda i: (0, 0)),
            pl.BlockSpec((D, 128), lambda i: (0, 0)),
            pl.BlockSpec((1, 128), lambda i: (0, 0)),
        ],
        out_specs=[
            pl.BlockSpec((BT, D), lambda i: (i, 0)),
            pl.BlockSpec((BT, D), lambda i: (i, 0)),
            pl.BlockSpec((BT, 128), lambda i: (i, 0)),
        ],
        out_shape=[
            jax.ShapeDtypeStruct((S, D), jnp.float32),
            jax.ShapeDtypeStruct((S, D), jnp.float32),
            jax.ShapeDtypeStruct((S, 128), jnp.float32),
        ],
    )(x2d, ctx, Wo, bo2, g2, b2v, ws_pad, bs_pad)

    lgf = lg_pad[:, :E].reshape(NW, TPW, E).transpose(0, 2, 1).reshape(-1)

    rpm, route, cnt_part, ps_part = _sc_router(lgf)
    pos, z_sorted, block_expert, counts_f, prob_sum = _sc_dispatch(
        route, cnt_part, ps_part, z2)

    w1b = W1.astype(jnp.bfloat16)
    w2b = W2.astype(jnp.bfloat16)
    y_sorted = pl.pallas_call(
        _k5_ffn,
        grid_spec=pltpu.PrefetchScalarGridSpec(
            num_scalar_prefetch=1,
            grid=(NB,),
            in_specs=[
                pl.BlockSpec((BS, D), lambda i, e: (i, 0)),
                pl.BlockSpec((1, D, DFF), lambda i, e: (e[i], 0, 0)),
                pl.BlockSpec((1, 1, DFF), lambda i, e: (e[i], 0, 0)),
                pl.BlockSpec((1, DFF, D), lambda i, e: (e[i], 0, 0)),
                pl.BlockSpec((1, 1, D), lambda i, e: (e[i], 0, 0)),
            ],
            out_specs=pl.BlockSpec((BS, D), lambda i, e: (i, 0)),
        ),
        out_shape=jax.ShapeDtypeStruct((PAD_T, D), jnp.float32),
    )(block_expert, z_sorted, w1b, b1.reshape(E, 1, DFF), w2b, b2.reshape(E, 1, D))

    out2d = _sc_combine(x2, y_sorted, pos, rpm)
    out = out2d.reshape(S, 1, D)
    return (out, counts_f[:E], prob_sum[:E],
            jnp.array(0, jnp.int32), rpm)


# speedup vs baseline: 2.9836x; 1.1430x over previous
import functools

import jax
import jax.numpy as jnp
from jax import lax
from jax.experimental import pallas as pl
from jax.experimental.pallas import tpu as pltpu
from jax.experimental.pallas import tpu_sc as plsc

S, D = 2048, 1024
H, DK = 16, 64
E, DFF = 8, 2048
BS = 128
NB = S // BS + E - 1
PAD_T = NB * BS
NW = 32
TPW = S // NW
_mesh = plsc.VectorSubcoreMesh(core_axis_name="c", subcore_axis_name="s")


def _k1_ln_qkv(x_ref, g_ref, b_ref, wq_ref, bq_ref, wk_ref, bk_ref,
               wv_ref, bv_ref, q_ref, k_ref, v_ref):
    xb = x_ref[...]
    mu = jnp.mean(xb, axis=1, keepdims=True)
    var = jnp.mean((xb - mu) ** 2, axis=1, keepdims=True)
    z = (xb - mu) * jax.lax.rsqrt(var + 1e-5) * g_ref[...] + b_ref[...]
    bt = z.shape[0]
    q = (jnp.dot(z, wq_ref[...]) + bq_ref[...]).reshape(bt, H, DK)
    k = (jnp.dot(z, wk_ref[...]) + bk_ref[...]).reshape(bt, H, DK)
    v = (jnp.dot(z, wv_ref[...]) + bv_ref[...]).reshape(bt, H, DK)
    q_ref[...] = jnp.transpose(q, (1, 0, 2))
    k_ref[...] = jnp.transpose(k, (1, 0, 2))
    v_ref[...] = jnp.transpose(v, (1, 0, 2))


def _k2_attn(q_ref, k_ref, v_ref, o_ref):
    q = q_ref[0]
    s = jax.lax.dot_general(q, k_ref[0],
                            (((1,), (1,)), ((), ()))) * (1.0 / 8.0)
    m = jnp.max(s, axis=1, keepdims=True)
    p = jnp.exp(s - m)
    l = jnp.sum(p, axis=1, keepdims=True)
    o_ref[0] = jnp.dot(p, v_ref[0]) / l


def _k3_post(x_ref, c_ref, wo_ref, bo_ref, g_ref, b_ref, ws_ref, bs_ref,
             x2_ref, z2_ref, lg_ref):
    c3 = jnp.transpose(c_ref[...], (1, 0, 2))
    ctx = c3.reshape(c3.shape[0], D)
    x2 = x_ref[...] + jnp.dot(ctx, wo_ref[...]) + bo_ref[...]
    x2_ref[...] = x2
    mu = jnp.mean(x2, axis=1, keepdims=True)
    var = jnp.mean((x2 - mu) ** 2, axis=1, keepdims=True)
    z2 = (x2 - mu) * jax.lax.rsqrt(var + 1e-5) * g_ref[...] + b_ref[...]
    z2_ref[...] = z2
    lg_ref[...] = jnp.dot(z2, ws_ref[...]) + bs_ref[...]


def _dyng(x, idx):
    return lax.gather(
        x, idx[:, None],
        lax.GatherDimensionNumbers(offset_dims=(), collapsed_slice_dims=(0,),
                                   start_index_map=(0,)),
        (1,), mode=lax.GatherScatterMode.PROMISE_IN_BOUNDS)


def _prefix16(x, lane):
    zero = x - x
    for k in (1, 2, 4, 8):
        g = _dyng(x, jnp.maximum(lane - k, 0))
        x = x + jnp.where(lane >= k, g, zero)
    return x


@functools.partial(
    pl.kernel, mesh=_mesh,
    compiler_params=pltpu.CompilerParams(needs_layout_passes=False),
    out_type=(
        jax.ShapeDtypeStruct((S,), jnp.float32),
        jax.ShapeDtypeStruct((S,), jnp.int32),
        jax.ShapeDtypeStruct((NW * 16,), jnp.int32),
        jax.ShapeDtypeStruct((NW * 16,), jnp.float32),
    ),
    scratch_types=[
        pltpu.VMEM((E * TPW,), jnp.float32),
        pltpu.VMEM((TPW,), jnp.float32),
        pltpu.VMEM((TPW,), jnp.int32),
        pltpu.VMEM((16,), jnp.int32),
        pltpu.VMEM((16,), jnp.float32),
    ],
)
def _sc_router(lgT_hbm, rpm_hbm, route_hbm, cnt_hbm, ps_hbm,
               lg_v, rpm_v, rt_v, cnt_v, ps_v):
    wid = lax.axis_index("s") * 2 + lax.axis_index("c")
    base = wid * TPW
    pltpu.sync_copy(lgT_hbm.at[pl.ds(wid * E * TPW, E * TPW)], lg_v)
    lane = lax.iota(jnp.int32, 16)
    last = jnp.full((16,), 15, jnp.int32)
    cnt_acc = [jnp.zeros((16,), jnp.int32) for _ in range(E)]
    ps_acc = [jnp.zeros((16,), jnp.float32) for _ in range(E)]
    for c in range(TPW // 16):
        l = [lg_v[pl.ds(e * TPW + c * 16, 16)] for e in range(E)]
        m = l[0]
        for e in range(1, E):
            m = jnp.maximum(m, l[e])
        ex = [jnp.exp(l[e] - m) for e in range(E)]
        sden = ex[0]
        for e in range(1, E):
            sden = sden + ex[e]
        rpm_c = 1.0 / sden
        best = jnp.zeros((16,), jnp.int32)
        bl = l[0]
        for e in range(1, E):
            flip = l[e] > bl
            bl = jnp.maximum(bl, l[e])
            best = jnp.where(flip, e, best)
        rpm_v[pl.ds(c * 16, 16)] = rpm_c
        rt_v[pl.ds(c * 16, 16)] = best
        for e in range(E):
            ps_acc[e] = ps_acc[e] + ex[e] * rpm_c
            cnt_acc[e] = cnt_acc[e] + jnp.where(best == e, 1, 0)
    cntv = jnp.zeros((16,), jnp.int32)
    psv = jnp.zeros((16,), jnp.float32)
    for e in range(E):
        tot_c = _dyng(_prefix16(cnt_acc[e], lane), last)
        cntv = jnp.where(lane == e, tot_c, cntv)
        tot_p = _dyng(_prefix16(ps_acc[e], lane), last)
        psv = jnp.where(lane == e, tot_p, psv)
    cnt_v[...] = cntv
    ps_v[...] = psv
    pltpu.sync_copy(rpm_v, rpm_hbm.at[pl.ds(base, TPW)])
    pltpu.sync_copy(rt_v, route_hbm.at[pl.ds(base, TPW)])
    pltpu.sync_copy(cnt_v, cnt_hbm.at[pl.ds(wid * 16, 16)])
    pltpu.sync_copy(ps_v, ps_hbm.at[pl.ds(wid * 16, 16)])


@functools.partial(
    pl.kernel, mesh=_mesh,
    compiler_params=pltpu.CompilerParams(needs_layout_passes=False),
    out_type=(
        jax.ShapeDtypeStruct((S,), jnp.int32),
        jax.ShapeDtypeStruct((PAD_T, D), jnp.float32),
        jax.ShapeDtypeStruct((32,), jnp.int32),
        jax.ShapeDtypeStruct((16,), jnp.float32),
        jax.ShapeDtypeStruct((16,), jnp.float32),
    ),
    scratch_types=[
        pltpu.VMEM((NW * 16,), jnp.int32),
        pltpu.VMEM((NW * 16,), jnp.float32),
        pltpu.VMEM((TPW,), jnp.int32),
        pltpu.VMEM((TPW,), jnp.int32),
        pltpu.VMEM((32,), jnp.int32),
        pltpu.VMEM((16,), jnp.float32),
        pltpu.VMEM((TPW, D), jnp.float32),
        pltpu.SemaphoreType.DMA,
    ],
)
def _sc_dispatch(route_hbm, cnt_hbm, ps_hbm, z2_hbm,
                 pos_hbm, zs_hbm, be_hbm, cf_hbm, pso_hbm,
                 cp_v, pp_v, rt_v, pos_v, be_v, f_v, z_v, sem):
    wid = lax.axis_index("s") * 2 + lax.axis_index("c")
    base = wid * TPW
    pltpu.sync_copy(cnt_hbm, cp_v)
    pltpu.sync_copy(route_hbm.at[pl.ds(base, TPW)], rt_v)
    lane = lax.iota(jnp.int32, 16)
    last = jnp.full((16,), 15, jnp.int32)
    total = jnp.zeros((16,), jnp.int32)
    before = jnp.zeros((16,), jnp.int32)
    for t in range(NW):
        row = cp_v[pl.ds(t * 16, 16)]
        total = total + row
        before = before + row * jnp.where(wid > t, 1, 0)
    nb = lax.shift_right_logical(total + (BS - 1), 7)
    cumnb = _prefix16(nb, lane)
    pstart = (cumnb - nb) * BS
    runv = pstart + before
    for c in range(TPW // 16):
        r = rt_v[pl.ds(c * 16, 16)]
        posc = jnp.zeros((16,), jnp.int32)
        addv = jnp.zeros((16,), jnp.int32)
        for e in range(E):
            mk = r == e
            inc = _prefix16(mk.astype(jnp.int32), lane)
            tot = _dyng(inc, last)
            base_e = _dyng(runv, jnp.full((16,), e, jnp.int32))
            posc = jnp.where(mk, base_e + inc - 1, posc)
            addv = addv + jnp.where(lane == e, tot, 0)
        runv = runv + addv
        pos_v[pl.ds(c * 16, 16)] = posc
    pltpu.sync_copy(pos_v, pos_hbm.at[pl.ds(base, TPW)])
    pltpu.sync_copy(z2_hbm.at[pl.ds(base, TPW)], z_v)
    pltpu.async_copy(z_v, zs_hbm.at[pos_v], sem).wait()

    @pl.when(wid == 0)
    def _():
        acc0 = jnp.zeros((16,), jnp.int32)
        acc1 = jnp.zeros((16,), jnp.int32)
        for e in range(E):
            ce = _dyng(cumnb, jnp.full((16,), e, jnp.int32))
            acc0 = acc0 + jnp.where(ce <= lane, 1, 0)
            acc1 = acc1 + jnp.where(ce <= lane + 16, 1, 0)
        be_v[pl.ds(0, 16)] = jnp.minimum(acc0, E - 1)
        be_v[pl.ds(16, 16)] = jnp.minimum(acc1, E - 1)
        pltpu.sync_copy(be_v, be_hbm)
        f_v[...] = total.astype(jnp.float32)
        pltpu.sync_copy(f_v, cf_hbm)
        pltpu.sync_copy(ps_hbm, pp_v)
        pstot = jnp.zeros((16,), jnp.float32)
        for t in range(NW):
            pstot = pstot + pp_v[pl.ds(t * 16, 16)]
        f_v[...] = pstot
        pltpu.sync_copy(f_v, pso_hbm)


@functools.partial(
    pl.kernel, mesh=_mesh,
    compiler_params=pltpu.CompilerParams(needs_layout_passes=False),
    out_type=jax.ShapeDtypeStruct((S, D), jnp.float32),
    scratch_types=[
        pltpu.VMEM((32, D), jnp.float32),
        pltpu.VMEM((32, D), jnp.float32),
        pltpu.VMEM((32,), jnp.int32),
        pltpu.VMEM((32,), jnp.float32),
        pltpu.SemaphoreType.DMA,
    ],
)
def _sc_combine(x2_hbm, ys_hbm, pos_hbm, rpm_hbm, out_hbm,
                xv, yv, pv, rv, sem):
    wid = lax.axis_index("s") * 2 + lax.axis_index("c")
    for c in range(TPW // 32):
        b2 = wid * TPW + c * 32
        pltpu.sync_copy(x2_hbm.at[pl.ds(b2, 32)], xv)
        pltpu.sync_copy(pos_hbm.at[pl.ds(b2, 32)], pv)
        pltpu.sync_copy(rpm_hbm.at[pl.ds(b2, 32)], rv)
        pltpu.async_copy(ys_hbm.at[pv], yv, sem).wait()

        def body(t, carry):
            sc = plsc.load_gather(rv, [jnp.zeros((16,), jnp.int32) + t])
            for d in range(D // 16):
                sl = pl.ds(d * 16, 16)
                xv[t, sl] = xv[t, sl] + sc * yv[t, sl]
            return carry

        lax.fori_loop(0, 32, body, 0)
        pltpu.sync_copy(xv, out_hbm.at[pl.ds(b2, 32)])


def _k5_ffn(e_ref, z_ref, w1_ref, b1_ref, w2_ref, b2_ref, y_ref):
    z = z_ref[...].astype(jnp.bfloat16)
    h = jnp.dot(z, w1_ref[0], preferred_element_type=jnp.float32) + b1_ref[0]
    h = jnp.maximum(h, 0.0).astype(jnp.bfloat16)
    y_ref[...] = jnp.dot(h, w2_ref[0], preferred_element_type=jnp.float32) + b2_ref[0]


def kernel(x, mask, ln1_g, ln1_b, ln2_g, ln2_b, Wq, bq, Wk, bk, Wv, bv,
           Wo, bo, Ws, bs, W1, b1, W2, b2):
    x2d = x.reshape(S, D)
    g1 = ln1_g.reshape(1, D); b1v = ln1_b.reshape(1, D)
    g2 = ln2_g.reshape(1, D); b2v = ln2_b.reshape(1, D)
    bq2 = bq.reshape(1, D); bk2 = bk.reshape(1, D)
    bv2 = bv.reshape(1, D); bo2 = bo.reshape(1, D)
    ws_pad = jnp.zeros((D, 128), jnp.float32).at[:, :E].set(Ws)
    bs_pad = jnp.zeros((1, 128), jnp.float32).at[0, :E].set(bs)

    BT = 256
    qkv = pl.pallas_call(
        _k1_ln_qkv,
        grid=(S // BT,),
        in_specs=[
            pl.BlockSpec((BT, D), lambda i: (i, 0)),
            pl.BlockSpec((1, D), lambda i: (0, 0)),
            pl.BlockSpec((1, D), lambda i: (0, 0)),
            pl.BlockSpec((D, D), lambda i: (0, 0)),
            pl.BlockSpec((1, D), lambda i: (0, 0)),
            pl.BlockSpec((D, D), lambda i: (0, 0)),
            pl.BlockSpec((1, D), lambda i: (0, 0)),
            pl.BlockSpec((D, D), lambda i: (0, 0)),
            pl.BlockSpec((1, D), lambda i: (0, 0)),
        ],
        out_specs=[pl.BlockSpec((H, BT, DK), lambda i: (0, i, 0))] * 3,
        out_shape=[jax.ShapeDtypeStruct((H, S, DK), jnp.float32)] * 3,
    )(x2d, g1, b1v, Wq, bq2, Wk, bk2, Wv, bv2)
    qh, kh, vh = qkv

    BQ = 512
    ctx3 = pl.pallas_call(
        _k2_attn,
        grid=(H, S // BQ),
        in_specs=[
            pl.BlockSpec((1, BQ, DK), lambda h, i: (h, i, 0)),
            pl.BlockSpec((1, S, DK), lambda h, i: (h, 0, 0)),
            pl.BlockSpec((1, S, DK), lambda h, i: (h, 0, 0)),
        ],
        out_specs=pl.BlockSpec((1, BQ, DK), lambda h, i: (h, i, 0)),
        out_shape=jax.ShapeDtypeStruct((H, S, DK), jnp.float32),
    )(qh, kh, vh)

    x2, z2, lg_pad = pl.pallas_call(
        _k3_post,
        grid=(S // BT,),
        in_specs=[
            pl.BlockSpec((BT, D), lambda i: (i, 0)),
            pl.BlockSpec((H, BT, DK), lambda i: (0, i, 0)),
            pl.BlockSpec((D, D), lambda i: (0, 0)),
            pl.BlockSpec((1, D), lambda i: (0, 0)),
            pl.BlockSpec((1, D), lambda i: (0, 0)),
            pl.BlockSpec((1, D), lambda i: (0, 0)),
            pl.BlockSpec((D, 128), lambda i: (0, 0)),
            pl.BlockSpec((1, 128), lambda i: (0, 0)),
        ],
        out_specs=[
            pl.BlockSpec((BT, D), lambda i: (i, 0)),
            pl.BlockSpec((BT, D), lambda i: (i, 0)),
            pl.BlockSpec((BT, 128), lambda i: (i, 0)),
        ],
        out_shape=[
            jax.ShapeDtypeStruct((S, D), jnp.float32),
            jax.ShapeDtypeStruct((S, D), jnp.float32),
            jax.ShapeDtypeStruct((S, 128), jnp.float32),
        ],
    )(x2d, ctx3, Wo, bo2, g2, b2v, ws_pad, bs_pad)

    lgf = lg_pad[:, :E].reshape(NW, TPW, E).transpose(0, 2, 1).reshape(-1)

    rpm, route, cnt_part, ps_part = _sc_router(lgf)
    pos, z_sorted, block_expert, counts_f, prob_sum = _sc_dispatch(
        route, cnt_part, ps_part, z2)

    w1b = W1.astype(jnp.bfloat16)
    w2b = W2.astype(jnp.bfloat16)
    y_sorted = pl.pallas_call(
        _k5_ffn,
        grid_spec=pltpu.PrefetchScalarGridSpec(
            num_scalar_prefetch=1,
            grid=(NB,),
            in_specs=[
                pl.BlockSpec((BS, D), lambda i, e: (i, 0)),
                pl.BlockSpec((1, D, DFF), lambda i, e: (e[i], 0, 0)),
                pl.BlockSpec((1, 1, DFF), lambda i, e: (e[i], 0, 0)),
                pl.BlockSpec((1, DFF, D), lambda i, e: (e[i], 0, 0)),
                pl.BlockSpec((1, 1, D), lambda i, e: (e[i], 0, 0)),
            ],
            out_specs=pl.BlockSpec((BS, D), lambda i, e: (i, 0)),
        ),
        out_shape=jax.ShapeDtypeStruct((PAD_T, D), jnp.float32),
    )(block_expert, z_sorted, w1b, b1.reshape(E, 1, DFF), w2b, b2.reshape(E, 1, D))

    out2d = _sc_combine(x2, y_sorted, pos, rpm)
    out = out2d.reshape(S, 1, D)
    return (out, counts_f[:E], prob_sum[:E],
            jnp.array(0, jnp.int32), rpm)


# speedup vs baseline: 2.9877x; 1.0014x over previous
import functools

import jax
import jax.numpy as jnp
from jax import lax
from jax.experimental import pallas as pl
from jax.experimental.pallas import tpu as pltpu
from jax.experimental.pallas import tpu_sc as plsc

S, D = 2048, 1024
H, DK = 16, 64
E, DFF = 8, 2048
BS = 128
NB = S // BS + E - 1
PAD_T = NB * BS
NW = 32
TPW = S // NW
_mesh = plsc.VectorSubcoreMesh(core_axis_name="c", subcore_axis_name="s")


def _k1_ln_qkv(x_ref, g_ref, b_ref, wq_ref, bq_ref, wk_ref, bk_ref,
               wv_ref, bv_ref, q_ref, k_ref, v_ref):
    xb = x_ref[...]
    mu = jnp.mean(xb, axis=1, keepdims=True)
    var = jnp.mean((xb - mu) ** 2, axis=1, keepdims=True)
    z = (xb - mu) * jax.lax.rsqrt(var + 1e-5) * g_ref[...] + b_ref[...]
    bt = z.shape[0]
    q = (jnp.dot(z, wq_ref[...]) + bq_ref[...]).reshape(bt, H, DK)
    k = (jnp.dot(z, wk_ref[...]) + bk_ref[...]).reshape(bt, H, DK)
    v = (jnp.dot(z, wv_ref[...]) + bv_ref[...]).reshape(bt, H, DK)
    q_ref[...] = jnp.transpose(q, (1, 0, 2))
    k_ref[...] = jnp.transpose(k, (1, 0, 2))
    v_ref[...] = jnp.transpose(v, (1, 0, 2))


def _k2_attn(q_ref, k_ref, v_ref, o_ref):
    q = q_ref[0]
    s = jax.lax.dot_general(q, k_ref[0],
                            (((1,), (1,)), ((), ()))) * (1.0 / 8.0)
    m = jnp.max(s, axis=1, keepdims=True)
    p = jnp.exp(s - m)
    l = jnp.sum(p, axis=1, keepdims=True)
    o_ref[0] = jnp.dot(p, v_ref[0]) / l


def _k3_post(x_ref, c_ref, wo_ref, bo_ref, g_ref, b_ref, ws_ref, bs_ref,
             x2_ref, z2_ref, lg_ref):
    c3 = jnp.transpose(c_ref[...], (1, 0, 2))
    ctx = c3.reshape(c3.shape[0], D)
    x2 = x_ref[...] + jnp.dot(ctx, wo_ref[...]) + bo_ref[...]
    x2_ref[...] = x2
    mu = jnp.mean(x2, axis=1, keepdims=True)
    var = jnp.mean((x2 - mu) ** 2, axis=1, keepdims=True)
    z2 = (x2 - mu) * jax.lax.rsqrt(var + 1e-5) * g_ref[...] + b_ref[...]
    z2_ref[...] = z2
    lg_ref[...] = jnp.dot(z2, ws_ref[...]) + bs_ref[...]


def _dyng(x, idx):
    return lax.gather(
        x, idx[:, None],
        lax.GatherDimensionNumbers(offset_dims=(), collapsed_slice_dims=(0,),
                                   start_index_map=(0,)),
        (1,), mode=lax.GatherScatterMode.PROMISE_IN_BOUNDS)


def _prefix16(x, lane):
    zero = x - x
    for k in (1, 2, 4, 8):
        g = _dyng(x, jnp.maximum(lane - k, 0))
        x = x + jnp.where(lane >= k, g, zero)
    return x


@functools.partial(
    pl.kernel, mesh=_mesh,
    compiler_params=pltpu.CompilerParams(needs_layout_passes=False, use_tc_tiling_on_sc=True),
    out_type=(
        jax.ShapeDtypeStruct((S,), jnp.float32),
        jax.ShapeDtypeStruct((S,), jnp.int32),
        jax.ShapeDtypeStruct((NW * 16,), jnp.int32),
        jax.ShapeDtypeStruct((NW * 16,), jnp.float32),
    ),
    scratch_types=[
        pltpu.VMEM((E * TPW,), jnp.float32),
        pltpu.VMEM((TPW,), jnp.float32),
        pltpu.VMEM((TPW,), jnp.int32),
        pltpu.VMEM((16,), jnp.int32),
        pltpu.VMEM((16,), jnp.float32),
    ],
)
def _sc_router(lgT_hbm, rpm_hbm, route_hbm, cnt_hbm, ps_hbm,
               lg_v, rpm_v, rt_v, cnt_v, ps_v):
    wid = lax.axis_index("s") * 2 + lax.axis_index("c")
    base = wid * TPW
    pltpu.sync_copy(lgT_hbm.at[pl.ds(wid * E * TPW, E * TPW)], lg_v)
    lane = lax.iota(jnp.int32, 16)
    last = jnp.full((16,), 15, jnp.int32)
    cnt_acc = [jnp.zeros((16,), jnp.int32) for _ in range(E)]
    ps_acc = [jnp.zeros((16,), jnp.float32) for _ in range(E)]
    for c in range(TPW // 16):
        l = [lg_v[pl.ds(e * TPW + c * 16, 16)] for e in range(E)]
        m = l[0]
        for e in range(1, E):
            m = jnp.maximum(m, l[e])
        ex = [jnp.exp(l[e] - m) for e in range(E)]
        sden = ex[0]
        for e in range(1, E):
            sden = sden + ex[e]
        rpm_c = 1.0 / sden
        best = jnp.zeros((16,), jnp.int32)
        bl = l[0]
        for e in range(1, E):
            flip = l[e] > bl
            bl = jnp.maximum(bl, l[e])
            best = jnp.where(flip, e, best)
        rpm_v[pl.ds(c * 16, 16)] = rpm_c
        rt_v[pl.ds(c * 16, 16)] = best
        for e in range(E):
            ps_acc[e] = ps_acc[e] + ex[e] * rpm_c
            cnt_acc[e] = cnt_acc[e] + jnp.where(best == e, 1, 0)
    cntv = jnp.zeros((16,), jnp.int32)
    psv = jnp.zeros((16,), jnp.float32)
    for e in range(E):
        tot_c = _dyng(_prefix16(cnt_acc[e], lane), last)
        cntv = jnp.where(lane == e, tot_c, cntv)
        tot_p = _dyng(_prefix16(ps_acc[e], lane), last)
        psv = jnp.where(lane == e, tot_p, psv)
    cnt_v[...] = cntv
    ps_v[...] = psv
    pltpu.sync_copy(rpm_v, rpm_hbm.at[pl.ds(base, TPW)])
    pltpu.sync_copy(rt_v, route_hbm.at[pl.ds(base, TPW)])
    pltpu.sync_copy(cnt_v, cnt_hbm.at[pl.ds(wid * 16, 16)])
    pltpu.sync_copy(ps_v, ps_hbm.at[pl.ds(wid * 16, 16)])


@functools.partial(
    pl.kernel, mesh=_mesh,
    compiler_params=pltpu.CompilerParams(needs_layout_passes=False, use_tc_tiling_on_sc=True),
    out_type=(
        jax.ShapeDtypeStruct((S,), jnp.int32),
        jax.ShapeDtypeStruct((PAD_T, D), jnp.float32),
        jax.ShapeDtypeStruct((32,), jnp.int32),
        jax.ShapeDtypeStruct((16,), jnp.float32),
        jax.ShapeDtypeStruct((16,), jnp.float32),
    ),
    scratch_types=[
        pltpu.VMEM((NW * 16,), jnp.int32),
        pltpu.VMEM((NW * 16,), jnp.float32),
        pltpu.VMEM((TPW,), jnp.int32),
        pltpu.VMEM((TPW,), jnp.int32),
        pltpu.VMEM((32,), jnp.int32),
        pltpu.VMEM((16,), jnp.float32),
        pltpu.VMEM((TPW, D), jnp.float32),
        pltpu.SemaphoreType.DMA,
    ],
)
def _sc_dispatch(route_hbm, cnt_hbm, ps_hbm, z2_hbm,
                 pos_hbm, zs_hbm, be_hbm, cf_hbm, pso_hbm,
                 cp_v, pp_v, rt_v, pos_v, be_v, f_v, z_v, sem):
    wid = lax.axis_index("s") * 2 + lax.axis_index("c")
    base = wid * TPW
    pltpu.sync_copy(cnt_hbm, cp_v)
    pltpu.sync_copy(route_hbm.at[pl.ds(base, TPW)], rt_v)
    lane = lax.iota(jnp.int32, 16)
    last = jnp.full((16,), 15, jnp.int32)
    total = jnp.zeros((16,), jnp.int32)
    before = jnp.zeros((16,), jnp.int32)
    for t in range(NW):
        row = cp_v[pl.ds(t * 16, 16)]
        total = total + row
        before = before + row * jnp.where(wid > t, 1, 0)
    nb = lax.shift_right_logical(total + (BS - 1), 7)
    cumnb = _prefix16(nb, lane)
    pstart = (cumnb - nb) * BS
    runv = pstart + before
    for c in range(TPW // 16):
        r = rt_v[pl.ds(c * 16, 16)]
        posc = jnp.zeros((16,), jnp.int32)
        addv = jnp.zeros((16,), jnp.int32)
        for e in range(E):
            mk = r == e
            inc = _prefix16(mk.astype(jnp.int32), lane)
            tot = _dyng(inc, last)
            base_e = _dyng(runv, jnp.full((16,), e, jnp.int32))
            posc = jnp.where(mk, base_e + inc - 1, posc)
            addv = addv + jnp.where(lane == e, tot, 0)
        runv = runv + addv
        pos_v[pl.ds(c * 16, 16)] = posc
    pltpu.sync_copy(pos_v, pos_hbm.at[pl.ds(base, TPW)])
    pltpu.sync_copy(z2_hbm.at[pl.ds(base, TPW)], z_v)
    pltpu.async_copy(z_v, zs_hbm.at[pos_v], sem).wait()

    @pl.when(wid == 0)
    def _():
        acc0 = jnp.zeros((16,), jnp.int32)
        acc1 = jnp.zeros((16,), jnp.int32)
        for e in range(E):
            ce = _dyng(cumnb, jnp.full((16,), e, jnp.int32))
            acc0 = acc0 + jnp.where(ce <= lane, 1, 0)
            acc1 = acc1 + jnp.where(ce <= lane + 16, 1, 0)
        be_v[pl.ds(0, 16)] = jnp.minimum(acc0, E - 1)
        be_v[pl.ds(16, 16)] = jnp.minimum(acc1, E - 1)
        pltpu.sync_copy(be_v, be_hbm)
        f_v[...] = total.astype(jnp.float32)
        pltpu.sync_copy(f_v, cf_hbm)
        pltpu.sync_copy(ps_hbm, pp_v)
        pstot = jnp.zeros((16,), jnp.float32)
        for t in range(NW):
            pstot = pstot + pp_v[pl.ds(t * 16, 16)]
        f_v[...] = pstot
        pltpu.sync_copy(f_v, pso_hbm)


@functools.partial(
    pl.kernel, mesh=_mesh,
    compiler_params=pltpu.CompilerParams(needs_layout_passes=False, use_tc_tiling_on_sc=True),
    out_type=jax.ShapeDtypeStruct((S, D), jnp.float32),
    scratch_types=[
        pltpu.VMEM((32, D), jnp.float32),
        pltpu.VMEM((32, D), jnp.float32),
        pltpu.VMEM((32,), jnp.int32),
        pltpu.VMEM((32,), jnp.float32),
        pltpu.SemaphoreType.DMA,
    ],
)
def _sc_combine(x2_hbm, ys_hbm, pos_hbm, rpm_hbm, out_hbm,
                xv, yv, pv, rv, sem):
    wid = lax.axis_index("s") * 2 + lax.axis_index("c")
    for c in range(TPW // 32):
        b2 = wid * TPW + c * 32
        pltpu.sync_copy(x2_hbm.at[pl.ds(b2, 32)], xv)
        pltpu.sync_copy(pos_hbm.at[pl.ds(b2, 32)], pv)
        pltpu.sync_copy(rpm_hbm.at[pl.ds(b2, 32)], rv)
        pltpu.async_copy(ys_hbm.at[pv], yv, sem).wait()

        def body(t, carry):
            sc = plsc.load_gather(rv, [jnp.zeros((16,), jnp.int32) + t])
            for d in range(D // 16):
                sl = pl.ds(d * 16, 16)
                xv[t, sl] = xv[t, sl] + sc * yv[t, sl]
            return carry

        lax.fori_loop(0, 32, body, 0)
        pltpu.sync_copy(xv, out_hbm.at[pl.ds(b2, 32)])


def _k5_ffn(e_ref, z_ref, w1_ref, b1_ref, w2_ref, b2_ref, y_ref):
    z = z_ref[...].astype(jnp.bfloat16)
    h = jnp.dot(z, w1_ref[0], preferred_element_type=jnp.float32) + b1_ref[0]
    h = jnp.maximum(h, 0.0).astype(jnp.bfloat16)
    y_ref[...] = jnp.dot(h, w2_ref[0], preferred_element_type=jnp.float32) + b2_ref[0]


def kernel(x, mask, ln1_g, ln1_b, ln2_g, ln2_b, Wq, bq, Wk, bk, Wv, bv,
           Wo, bo, Ws, bs, W1, b1, W2, b2):
    x2d = x.reshape(S, D)
    g1 = ln1_g.reshape(1, D); b1v = ln1_b.reshape(1, D)
    g2 = ln2_g.reshape(1, D); b2v = ln2_b.reshape(1, D)
    bq2 = bq.reshape(1, D); bk2 = bk.reshape(1, D)
    bv2 = bv.reshape(1, D); bo2 = bo.reshape(1, D)
    ws_pad = jnp.zeros((D, 128), jnp.float32).at[:, :E].set(Ws)
    bs_pad = jnp.zeros((1, 128), jnp.float32).at[0, :E].set(bs)

    BT = 256
    qkv = pl.pallas_call(
        _k1_ln_qkv,
        grid=(S // BT,),
        in_specs=[
            pl.BlockSpec((BT, D), lambda i: (i, 0)),
            pl.BlockSpec((1, D), lambda i: (0, 0)),
            pl.BlockSpec((1, D), lambda i: (0, 0)),
            pl.BlockSpec((D, D), lambda i: (0, 0)),
            pl.BlockSpec((1, D), lambda i: (0, 0)),
            pl.BlockSpec((D, D), lambda i: (0, 0)),
            pl.BlockSpec((1, D), lambda i: (0, 0)),
            pl.BlockSpec((D, D), lambda i: (0, 0)),
            pl.BlockSpec((1, D), lambda i: (0, 0)),
        ],
        out_specs=[pl.BlockSpec((H, BT, DK), lambda i: (0, i, 0))] * 3,
        out_shape=[jax.ShapeDtypeStruct((H, S, DK), jnp.float32)] * 3,
    )(x2d, g1, b1v, Wq, bq2, Wk, bk2, Wv, bv2)
    qh, kh, vh = qkv

    BQ = 512
    ctx3 = pl.pallas_call(
        _k2_attn,
        grid=(H, S // BQ),
        in_specs=[
            pl.BlockSpec((1, BQ, DK), lambda h, i: (h, i, 0)),
            pl.BlockSpec((1, S, DK), lambda h, i: (h, 0, 0)),
            pl.BlockSpec((1, S, DK), lambda h, i: (h, 0, 0)),
        ],
        out_specs=pl.BlockSpec((1, BQ, DK), lambda h, i: (h, i, 0)),
        out_shape=jax.ShapeDtypeStruct((H, S, DK), jnp.float32),
    )(qh, kh, vh)

    x2, z2, lg_pad = pl.pallas_call(
        _k3_post,
        grid=(S // BT,),
        in_specs=[
            pl.BlockSpec((BT, D), lambda i: (i, 0)),
            pl.BlockSpec((H, BT, DK), lambda i: (0, i, 0)),
            pl.BlockSpec((D, D), lambda i: (0, 0)),
            pl.BlockSpec((1, D), lambda i: (0, 0)),
            pl.BlockSpec((1, D), lambda i: (0, 0)),
            pl.BlockSpec((1, D), lambda i: (0, 0)),
            pl.BlockSpec((D, 128), lambda i: (0, 0)),
            pl.BlockSpec((1, 128), lambda i: (0, 0)),
        ],
        out_specs=[
            pl.BlockSpec((BT, D), lambda i: (i, 0)),
            pl.BlockSpec((BT, D), lambda i: (i, 0)),
            pl.BlockSpec((BT, 128), lambda i: (i, 0)),
        ],
        out_shape=[
            jax.ShapeDtypeStruct((S, D), jnp.float32),
            jax.ShapeDtypeStruct((S, D), jnp.float32),
            jax.ShapeDtypeStruct((S, 128), jnp.float32),
        ],
    )(x2d, ctx3, Wo, bo2, g2, b2v, ws_pad, bs_pad)

    lgf = lg_pad[:, :E].reshape(NW, TPW, E).transpose(0, 2, 1).reshape(-1)

    rpm, route, cnt_part, ps_part = _sc_router(lgf)
    pos, z_sorted, block_expert, counts_f, prob_sum = _sc_dispatch(
        route, cnt_part, ps_part, z2)

    w1b = W1.astype(jnp.bfloat16)
    w2b = W2.astype(jnp.bfloat16)
    y_sorted = pl.pallas_call(
        _k5_ffn,
        grid_spec=pltpu.PrefetchScalarGridSpec(
            num_scalar_prefetch=1,
            grid=(NB,),
            in_specs=[
                pl.BlockSpec((BS, D), lambda i, e: (i, 0)),
                pl.BlockSpec((1, D, DFF), lambda i, e: (e[i], 0, 0)),
                pl.BlockSpec((1, 1, DFF), lambda i, e: (e[i], 0, 0)),
                pl.BlockSpec((1, DFF, D), lambda i, e: (e[i], 0, 0)),
                pl.BlockSpec((1, 1, D), lambda i, e: (e[i], 0, 0)),
            ],
            out_specs=pl.BlockSpec((BS, D), lambda i, e: (i, 0)),
        ),
        out_shape=jax.ShapeDtypeStruct((PAD_T, D), jnp.float32),
    )(block_expert, z_sorted, w1b, b1.reshape(E, 1, DFF), w2b, b2.reshape(E, 1, D))

    out2d = _sc_combine(x2, y_sorted, pos, rpm)
    out = out2d.reshape(S, 1, D)
    return (out, counts_f[:E], prob_sum[:E],
            jnp.array(0, jnp.int32), rpm)
